# Initial kernel scaffold; baseline (speedup 1.0000x reference)
import jax
import jax.numpy as jnp
from jax.experimental import pallas as pl


def kernel(x_nchw, w1, b1, w2, b2, w3f, b3f, w4, b4, w5, b5, a1, a2, a3, a4):
    raise NotImplementedError("write your pallas kernel here")


# speedup vs baseline: 127.2067x; 127.2067x over previous
import jax
import jax.numpy as jnp
import numpy as np
from jax import lax
from jax.experimental import pallas as pl
from jax.experimental.pallas import tpu as pltpu


def _prelu(y, a):
    return jnp.where(y > 0, y, y * a)


def _pool_rows(h, n_out):
    B, H, L = h.shape
    y = h.reshape(B, H // 2, 2, L)
    even = y[:, :, 0, :]
    odd = y[:, :, 1, :]
    return jnp.maximum(jnp.maximum(even[:, :n_out], odd[:, :n_out]),
                       even[:, 1:n_out + 1])


def _rnet_kernel(x_ref, w1_ref, b1_ref, a1_ref, w2_ref, b2_ref, a2_ref,
                 w3_ref, b3_ref, a3_ref, w4_ref, b4_ref, a4_ref,
                 w5_ref, b5_ref, o_ref):
    B = x_ref.shape[0]
    x = x_ref[...]

    acc = None
    for dy in range(3):
        lhs = x[:, dy:dy + 22, :].reshape(B * 22, 192)
        t = jnp.dot(lhs, w1_ref[dy], preferred_element_type=jnp.float32)
        acc = t if acc is None else acc + t
    h = _prelu(acc + b1_ref[...], a1_ref[0]).reshape(B, 22, 704)

    r = _pool_rows(h, 10)
    cols = []
    for j in range(10):
        blk = [lax.slice(r, (0, 0, (2 * j + dx) * 32),
                         (B, 10, (2 * j + dx) * 32 + 32)) for dx in range(3)]
        cols.append(jnp.maximum(jnp.maximum(blk[0], blk[1]), blk[2]))
    p1 = jnp.concatenate(cols, axis=2)

    acc2 = None
    for dy in range(3):
        lhs = p1[:, dy:dy + 8, :].reshape(B * 8, 320)
        t = jnp.dot(lhs, w2_ref[dy], preferred_element_type=jnp.float32)
        acc2 = t if acc2 is None else acc2 + t
    h2 = _prelu(acc2 + b2_ref[...], a2_ref[0]).reshape(B, 8, 512)

    r2 = _pool_rows(h2, 3)
    cols2 = []
    for j in range(3):
        blk = [lax.slice(r2, (0, 0, (2 * j + dx) * 64),
                         (B, 3, (2 * j + dx) * 64 + 64)) for dx in range(3)]
        cols2.append(jnp.maximum(jnp.maximum(blk[0], blk[1]), blk[2]))
    p2 = jnp.concatenate(cols2, axis=2)

    acc3 = None
    for hrow in range(3):
        t = jnp.dot(p2[:, hrow, :], w3_ref[hrow],
                    preferred_element_type=jnp.float32)
        acc3 = t if acc3 is None else acc3 + t
    h3 = _prelu(acc3 + b3_ref[...], a3_ref[0])
    h4 = _prelu(jnp.dot(h3, w4_ref[...], preferred_element_type=jnp.float32)
                + b4_ref[...], a4_ref[0])
    o_ref[...] = (jnp.dot(h4, w5_ref[...], preferred_element_type=jnp.float32)
                  + b5_ref[...])


def _banded(sel, wv):
    t = jnp.einsum('wjd,ydco->ywcjo', sel, wv)
    d, wi, ci, jo, co = t.shape
    return t.reshape(d, wi * ci, jo * co)


def _sel(wi_n, jo_n):
    s = np.zeros((wi_n, jo_n, 3), np.float32)
    for j in range(jo_n):
        for dx in range(3):
            s[j + dx, j, dx] = 1.0
    return s


_SEL1 = _sel(24, 22)
_SEL2 = _sel(10, 8)


@jax.jit
def kernel(x_nchw, w1, b1, w2, b2, w3f, b3f, w4, b4, w5, b5, a1, a2, a3, a4):
    N = x_nchw.shape[0]
    x = jnp.transpose(x_nchw, (0, 2, 3, 1)).astype(jnp.float32)
    x = jnp.pad(x, ((0, 0), (0, 0), (0, 0), (0, 5))).reshape(N, 24, 192)

    w1b = _banded(_SEL1, w1.reshape(3, 3, 8, 32))
    w2b = _banded(_SEL2, w2.reshape(3, 3, 32, 64))
    b1t = jnp.tile(b1, 22).reshape(1, 704)
    b2t = jnp.tile(b2, 8).reshape(1, 512)
    w3r = w3f.reshape(3, 192, 256)

    B = 32
    Np = ((N + B - 1) // B) * B
    if Np != N:
        x = jnp.pad(x, ((0, Np - N), (0, 0), (0, 0)))

    smem = pl.BlockSpec(memory_space=pltpu.MemorySpace.SMEM)
    full = lambda shape: pl.BlockSpec(shape, lambda i: (0,) * len(shape))
    out = pl.pallas_call(
        _rnet_kernel,
        out_shape=jax.ShapeDtypeStruct((Np, 15), jnp.float32),
        grid=(Np // B,),
        in_specs=[
            pl.BlockSpec((B, 24, 192), lambda i: (i, 0, 0)),
            full((3, 192, 704)), full((1, 704)), smem,
            full((3, 320, 512)), full((1, 512)), smem,
            full((3, 192, 256)), full((1, 256)), smem,
            full((256, 128)), full((1, 128)), smem,
            full((128, 15)), full((1, 15)),
        ],
        out_specs=pl.BlockSpec((B, 15), lambda i: (i, 0)),
        compiler_params=pltpu.CompilerParams(
            dimension_semantics=("parallel",)),
    )(x, w1b, b1t, a1, w2b, b2t, a2, w3r, b3f.reshape(1, 256), a3,
      w4, b4.reshape(1, 128), a4, w5, b5.reshape(1, 15))
    return out[:N] if Np != N else out


# speedup vs baseline: 167.5782x; 1.3174x over previous
import jax
import jax.numpy as jnp
import numpy as np
from jax import lax
from jax.experimental import pallas as pl
from jax.experimental.pallas import tpu as pltpu


def _prelu(y, a):
    return jnp.where(y > 0, y, y * a)


def _max3(a, b, c):
    return jnp.maximum(jnp.maximum(a, b), c)


def _shl(v, lanes):
    n = v.shape[-1]
    return jnp.pad(lax.slice_in_dim(v, lanes, n, axis=2),
                   ((0, 0), (0, 0), (0, lanes)))


def _rnet_kernel(x_ref, w1_ref, b1_ref, a1_ref, w2_ref, b2_ref, a2_ref,
                 w3_ref, b3_ref, a3_ref, w4_ref, b4_ref, a4_ref,
                 w5_ref, b5_ref, o_ref):
    B = x_ref.shape[3]
    xq = [x_ref[0, q] for q in range(4)]

    h4 = []
    for q in range(4):
        nrows = 6 if q < 2 else 5
        acc = None
        for dy in range(3):
            r = q + dy
            lhs = xq[r % 4][r // 4:r // 4 + nrows].reshape(nrows * B, 192)
            t = jnp.dot(lhs, w1_ref[dy], preferred_element_type=jnp.float32)
            acc = t if acc is None else acc + t
        h4.append(_prelu(acc + b1_ref[...], a1_ref[0]).reshape(nrows, B, 704))

    rp_e = _max3(h4[0][:5], h4[1][:5], h4[2][:5])
    rp_o = _max3(h4[2][:5], h4[3][:5], h4[0][1:6])
    m_e = _max3(rp_e, _shl(rp_e, 32), _shl(rp_e, 64))
    m_o = _max3(rp_o, _shl(rp_o, 32), _shl(rp_o, 64))

    def conv2(slices):
        acc = None
        for dy, lhs in enumerate(slices):
            t = jnp.dot(lhs.reshape(4 * B, 704), w2_ref[dy],
                        preferred_element_type=jnp.float32)
            acc = t if acc is None else acc + t
        return _prelu(acc + b2_ref[...], a2_ref[0]).reshape(4, B, 512)

    h2e = conv2([m_e[0:4], m_o[0:4], m_e[1:5]])
    h2o = conv2([m_o[0:4], m_e[1:5], m_o[1:5]])

    r2 = _max3(h2e[0:3], h2o[0:3], h2e[1:4])
    m2 = _max3(r2, _shl(r2, 64), _shl(r2, 128))

    acc3 = None
    for hrow in range(3):
        t = jnp.dot(m2[hrow], w3_ref[hrow], preferred_element_type=jnp.float32)
        acc3 = t if acc3 is None else acc3 + t
    h3 = _prelu(acc3 + b3_ref[...], a3_ref[0])
    h4_ = _prelu(jnp.dot(h3, w4_ref[...], preferred_element_type=jnp.float32)
                 + b4_ref[...], a4_ref[0])
    o_ref[...] = (jnp.dot(h4_, w5_ref[...], preferred_element_type=jnp.float32)
                  + b5_ref[...])


def _banded(sel, wv):
    t = jnp.einsum('wjd,ydco->ywcjo', sel, wv)
    d, wi, ci, jo, co = t.shape
    return t.reshape(d, wi * ci, jo * co)


def _sel(wi_n, jo_n, stride):
    s = np.zeros((wi_n, jo_n, 3), np.float32)
    for j in range(jo_n):
        for dx in range(3):
            s[stride * (j + dx), j, dx] = 1.0
    return s


_SEL1 = _sel(24, 22, 1)
_SEL2 = _sel(22, 8, 2)
_P3 = np.zeros((8, 3), np.float32)
for _w in range(3):
    _P3[2 * _w, _w] = 1.0


@jax.jit
def kernel(x_nchw, w1, b1, w2, b2, w3f, b3f, w4, b4, w5, b5, a1, a2, a3, a4):
    N = x_nchw.shape[0]
    B = 32
    Np = ((N + B - 1) // B) * B
    x = jnp.transpose(x_nchw, (0, 2, 3, 1)).astype(jnp.float32)
    x = jnp.pad(x, ((0, 0), (0, 0), (0, 0), (0, 5))).reshape(N, 24, 192)
    if Np != N:
        x = jnp.pad(x, ((0, Np - N), (0, 0), (0, 0)))
    x = jnp.transpose(x.reshape(Np // B, B, 6, 4, 192), (0, 3, 2, 1, 4))

    w1b = _banded(jnp.asarray(_SEL1), w1.reshape(3, 3, 8, 32))
    w2b = _banded(jnp.asarray(_SEL2), w2.reshape(3, 3, 32, 64))
    w3r = jnp.einsum('vw,hwco->hvco', jnp.asarray(_P3),
                     w3f.reshape(3, 3, 64, 256)).reshape(3, 512, 256)
    b1t = jnp.tile(b1, 22).reshape(1, 704)
    b2t = jnp.tile(b2, 8).reshape(1, 512)

    smem = pl.BlockSpec(memory_space=pltpu.MemorySpace.SMEM)
    full = lambda shape: pl.BlockSpec(shape, lambda i: (0,) * len(shape))
    out = pl.pallas_call(
        _rnet_kernel,
        out_shape=jax.ShapeDtypeStruct((Np, 15), jnp.float32),
        grid=(Np // B,),
        in_specs=[
            pl.BlockSpec((1, 4, 6, B, 192), lambda i: (i, 0, 0, 0, 0)),
            full((3, 192, 704)), full((1, 704)), smem,
            full((3, 704, 512)), full((1, 512)), smem,
            full((3, 512, 256)), full((1, 256)), smem,
            full((256, 128)), full((1, 128)), smem,
            full((128, 15)), full((1, 15)),
        ],
        out_specs=pl.BlockSpec((B, 15), lambda i: (i, 0)),
        compiler_params=pltpu.CompilerParams(
            dimension_semantics=("parallel",)),
    )(x, w1b, b1t, a1, w2b, b2t, a2, w3r, b3f.reshape(1, 256), a3,
      w4, b4.reshape(1, 128), a4, w5, b5.reshape(1, 15))
    return out[:N] if Np != N else out


# speedup vs baseline: 220.8435x; 1.3179x over previous
import jax
import jax.numpy as jnp
import numpy as np
from jax import lax
from jax.experimental import pallas as pl
from jax.experimental.pallas import tpu as pltpu


def _prelu(y, a):
    return jnp.where(y > 0, y, y * a)


def _max3(a, b, c):
    return jnp.maximum(jnp.maximum(a, b), c)


def _shl(v, lanes):
    n = v.shape[-1]
    return jnp.pad(lax.slice_in_dim(v, lanes, n, axis=2),
                   ((0, 0), (0, 0), (0, lanes)))


def _rnet_kernel(x_ref, w1_ref, b1_ref, a1_ref, w2_ref, b2_ref, a2_ref,
                 w3_ref, b3_ref, a3_ref, w4_ref, b4_ref, a4_ref,
                 w5_ref, b5_ref, o_ref):
    B = x_ref.shape[3]
    xq = [x_ref[0, q] for q in range(4)]

    h4 = []
    for q in range(4):
        nrows = 6 if q < 2 else 5
        acc = None
        for dy in range(3):
            r = q + dy
            lhs = xq[r % 4][r // 4:r // 4 + nrows].reshape(nrows * B, 192)
            t = jnp.dot(lhs, w1_ref[dy], preferred_element_type=jnp.float32)
            acc = t if acc is None else acc + t
        h4.append(_prelu(acc + b1_ref[...], a1_ref[0]).reshape(nrows, B, 704))

    rp_e = _max3(h4[0][:5], h4[1][:5], h4[2][:5])
    rp_o = _max3(h4[2][:5], h4[3][:5], h4[0][1:6])
    m_e = _max3(rp_e, _shl(rp_e, 32), _shl(rp_e, 64))
    m_o = _max3(rp_o, _shl(rp_o, 32), _shl(rp_o, 64))

    def conv2(slices):
        acc = None
        for dy, lhs in enumerate(slices):
            t = jnp.dot(lhs.reshape(4 * B, 704), w2_ref[dy],
                        preferred_element_type=jnp.float32)
            acc = t if acc is None else acc + t
        return _prelu(acc + b2_ref[...], a2_ref[0]).reshape(4, B, 512)

    h2e = conv2([m_e[0:4], m_o[0:4], m_e[1:5]])
    h2o = conv2([m_o[0:4], m_e[1:5], m_o[1:5]])

    r2 = _max3(h2e[0:3], h2o[0:3], h2e[1:4])
    m2 = _max3(r2, _shl(r2, 64), _shl(r2, 128))

    acc3 = None
    for hrow in range(3):
        t = jnp.dot(m2[hrow], w3_ref[hrow], preferred_element_type=jnp.float32)
        acc3 = t if acc3 is None else acc3 + t
    h3 = _prelu(acc3 + b3_ref[...], a3_ref[0])
    h4_ = _prelu(jnp.dot(h3, w4_ref[...], preferred_element_type=jnp.float32)
                 + b4_ref[...], a4_ref[0])
    o_ref[...] = (jnp.dot(h4_, w5_ref[...], preferred_element_type=jnp.float32)
                  + b5_ref[...])


def _banded(sel, wv):
    t = jnp.einsum('wjd,ydco->ywcjo', sel, wv)
    d, wi, ci, jo, co = t.shape
    return t.reshape(d, wi * ci, jo * co)


def _sel(wi_n, jo_n, stride):
    s = np.zeros((wi_n, jo_n, 3), np.float32)
    for j in range(jo_n):
        for dx in range(3):
            s[stride * (j + dx), j, dx] = 1.0
    return s


_SEL1 = _sel(24, 22, 1)
_SEL2 = _sel(22, 8, 2)
_P3 = np.zeros((8, 3), np.float32)
for _w in range(3):
    _P3[2 * _w, _w] = 1.0


@jax.jit
def kernel(x_nchw, w1, b1, w2, b2, w3f, b3f, w4, b4, w5, b5, a1, a2, a3, a4):
    N = x_nchw.shape[0]
    B = 64
    Np = ((N + B - 1) // B) * B
    x = jnp.transpose(x_nchw, (0, 2, 3, 1)).astype(jnp.float32)
    x = jnp.pad(x, ((0, 0), (0, 0), (0, 0), (0, 5))).reshape(N, 24, 192)
    if Np != N:
        x = jnp.pad(x, ((0, Np - N), (0, 0), (0, 0)))
    x = jnp.transpose(x.reshape(Np // B, B, 6, 4, 192), (0, 3, 2, 1, 4))

    w1b = _banded(jnp.asarray(_SEL1), w1.reshape(3, 3, 8, 32))
    w2b = _banded(jnp.asarray(_SEL2), w2.reshape(3, 3, 32, 64))
    w3r = jnp.einsum('vw,hwco->hvco', jnp.asarray(_P3),
                     w3f.reshape(3, 3, 64, 256)).reshape(3, 512, 256)
    b1t = jnp.tile(b1, 22).reshape(1, 704)
    b2t = jnp.tile(b2, 8).reshape(1, 512)

    smem = pl.BlockSpec(memory_space=pltpu.MemorySpace.SMEM)
    full = lambda shape: pl.BlockSpec(shape, lambda i: (0,) * len(shape))
    out = pl.pallas_call(
        _rnet_kernel,
        out_shape=jax.ShapeDtypeStruct((Np, 15), jnp.float32),
        grid=(Np // B,),
        in_specs=[
            pl.BlockSpec((1, 4, 6, B, 192), lambda i: (i, 0, 0, 0, 0)),
            full((3, 192, 704)), full((1, 704)), smem,
            full((3, 704, 512)), full((1, 512)), smem,
            full((3, 512, 256)), full((1, 256)), smem,
            full((256, 128)), full((1, 128)), smem,
            full((128, 15)), full((1, 15)),
        ],
        out_specs=pl.BlockSpec((B, 15), lambda i: (i, 0)),
        compiler_params=pltpu.CompilerParams(
            dimension_semantics=("parallel",)),
    )(x, w1b, b1t, a1, w2b, b2t, a2, w3r, b3f.reshape(1, 256), a3,
      w4, b4.reshape(1, 128), a4, w5, b5.reshape(1, 15))
    return out[:N] if Np != N else out


# speedup vs baseline: 266.7843x; 1.2080x over previous
import jax
import jax.numpy as jnp
import numpy as np
from jax import lax
from jax.experimental import pallas as pl
from jax.experimental.pallas import tpu as pltpu


def _prelu(y, a):
    return jnp.where(y > 0, y, y * a)


def _max3(a, b, c):
    return jnp.maximum(jnp.maximum(a, b), c)


def _shl(v, lanes):
    n = v.shape[-1]
    return jnp.pad(lax.slice_in_dim(v, lanes, n, axis=2),
                   ((0, 0), (0, 0), (0, lanes)))


def _rnet_kernel(x_ref, w1_ref, b1_ref, a1_ref, w2_ref, b2_ref, a2_ref,
                 w3_ref, b3_ref, a3_ref, w4_ref, b4_ref, a4_ref,
                 w5_ref, b5_ref, o_ref):
    B = x_ref.shape[3]
    xq = [x_ref[0, q] for q in range(4)]

    h4 = []
    for q in range(4):
        nrows = 6 if q < 2 else 5
        acc = None
        for dy in range(3):
            r = q + dy
            lhs = xq[r % 4][r // 4:r // 4 + nrows].reshape(nrows * B, 192)
            t = jnp.dot(lhs, w1_ref[dy], preferred_element_type=jnp.float32)
            acc = t if acc is None else acc + t
        h4.append(_prelu(acc + b1_ref[...], a1_ref[0]).reshape(nrows, B, 704))

    rp_e = _max3(h4[0][:5], h4[1][:5], h4[2][:5])
    rp_o = _max3(h4[2][:5], h4[3][:5], h4[0][1:6])
    m_e = _max3(rp_e, _shl(rp_e, 32), _shl(rp_e, 64))
    m_o = _max3(rp_o, _shl(rp_o, 32), _shl(rp_o, 64))

    def conv2(slices):
        acc = None
        for dy, lhs in enumerate(slices):
            t = jnp.dot(lhs.reshape(4 * B, 704), w2_ref[dy],
                        preferred_element_type=jnp.float32)
            acc = t if acc is None else acc + t
        return _prelu(acc + b2_ref[...], a2_ref[0]).reshape(4, B, 512)

    h2e = conv2([m_e[0:4], m_o[0:4], m_e[1:5]])
    h2o = conv2([m_o[0:4], m_e[1:5], m_o[1:5]])

    r2 = _max3(h2e[0:3], h2o[0:3], h2e[1:4])
    m2 = _max3(r2, _shl(r2, 64), _shl(r2, 128))

    acc3 = None
    for hrow in range(3):
        t = jnp.dot(m2[hrow], w3_ref[hrow], preferred_element_type=jnp.float32)
        acc3 = t if acc3 is None else acc3 + t
    h3 = _prelu(acc3 + b3_ref[...], a3_ref[0])
    h4_ = _prelu(jnp.dot(h3, w4_ref[...], preferred_element_type=jnp.float32)
                 + b4_ref[...], a4_ref[0])
    o_ref[...] = (jnp.dot(h4_, w5_ref[...], preferred_element_type=jnp.float32)
                  + b5_ref[...])


def _banded(sel, wv):
    t = jnp.einsum('wjd,ydco->ywcjo', sel, wv)
    d, wi, ci, jo, co = t.shape
    return t.reshape(d, wi * ci, jo * co)


def _sel(wi_n, jo_n, stride):
    s = np.zeros((wi_n, jo_n, 3), np.float32)
    for j in range(jo_n):
        for dx in range(3):
            s[stride * (j + dx), j, dx] = 1.0
    return s


_SEL1 = _sel(24, 22, 1)
_SEL2 = _sel(22, 8, 2)
_P3 = np.zeros((8, 3), np.float32)
for _w in range(3):
    _P3[2 * _w, _w] = 1.0


@jax.jit
def kernel(x_nchw, w1, b1, w2, b2, w3f, b3f, w4, b4, w5, b5, a1, a2, a3, a4):
    N = x_nchw.shape[0]
    B = 128
    Np = ((N + B - 1) // B) * B
    x = jnp.transpose(x_nchw, (0, 2, 3, 1)).astype(jnp.float32)
    x = jnp.pad(x, ((0, 0), (0, 0), (0, 0), (0, 5))).reshape(N, 24, 192)
    if Np != N:
        x = jnp.pad(x, ((0, Np - N), (0, 0), (0, 0)))
    x = jnp.transpose(x.reshape(Np // B, B, 6, 4, 192), (0, 3, 2, 1, 4))

    w1b = _banded(jnp.asarray(_SEL1), w1.reshape(3, 3, 8, 32))
    w2b = _banded(jnp.asarray(_SEL2), w2.reshape(3, 3, 32, 64))
    w3r = jnp.einsum('vw,hwco->hvco', jnp.asarray(_P3),
                     w3f.reshape(3, 3, 64, 256)).reshape(3, 512, 256)
    b1t = jnp.tile(b1, 22).reshape(1, 704)
    b2t = jnp.tile(b2, 8).reshape(1, 512)

    smem = pl.BlockSpec(memory_space=pltpu.MemorySpace.SMEM)
    full = lambda shape: pl.BlockSpec(shape, lambda i: (0,) * len(shape))
    out = pl.pallas_call(
        _rnet_kernel,
        out_shape=jax.ShapeDtypeStruct((Np, 15), jnp.float32),
        grid=(Np // B,),
        in_specs=[
            pl.BlockSpec((1, 4, 6, B, 192), lambda i: (i, 0, 0, 0, 0)),
            full((3, 192, 704)), full((1, 704)), smem,
            full((3, 704, 512)), full((1, 512)), smem,
            full((3, 512, 256)), full((1, 256)), smem,
            full((256, 128)), full((1, 128)), smem,
            full((128, 15)), full((1, 15)),
        ],
        out_specs=pl.BlockSpec((B, 15), lambda i: (i, 0)),
        compiler_params=pltpu.CompilerParams(
            dimension_semantics=("parallel",)),
    )(x, w1b, b1t, a1, w2b, b2t, a2, w3r, b3f.reshape(1, 256), a3,
      w4, b4.reshape(1, 128), a4, w5, b5.reshape(1, 15))
    return out[:N] if Np != N else out


# speedup vs baseline: 275.0884x; 1.0311x over previous
import jax
import jax.numpy as jnp
import numpy as np
from jax import lax
from jax.experimental import pallas as pl
from jax.experimental.pallas import tpu as pltpu


def _prelu(y, a):
    return jnp.where(y > 0, y, y * a)


def _max3(a, b, c):
    return jnp.maximum(jnp.maximum(a, b), c)


def _shl(v, lanes):
    n = v.shape[-1]
    return jnp.pad(lax.slice_in_dim(v, lanes, n, axis=2),
                   ((0, 0), (0, 0), (0, lanes)))


def _rnet_kernel(x_ref, w1_ref, b1_ref, a1_ref, w2_ref, b2_ref, a2_ref,
                 w3_ref, b3_ref, a3_ref, w4_ref, b4_ref, a4_ref,
                 w5_ref, b5_ref, o_ref):
    B = x_ref.shape[3]
    xq = [x_ref[0, q] for q in range(4)]

    h4 = []
    for q in range(4):
        nrows = 6 if q < 2 else 5
        acc = None
        for dy in range(3):
            r = q + dy
            lhs = xq[r % 4][r // 4:r // 4 + nrows].reshape(nrows * B, 192)
            t = jnp.dot(lhs, w1_ref[dy], preferred_element_type=jnp.float32)
            acc = t if acc is None else acc + t
        h4.append(_prelu(acc + b1_ref[...], a1_ref[0]).reshape(nrows, B, 704))

    rp_e = _max3(h4[0][:5], h4[1][:5], h4[2][:5])
    rp_o = _max3(h4[2][:5], h4[3][:5], h4[0][1:6])
    m_e = _max3(rp_e, _shl(rp_e, 32), _shl(rp_e, 64))
    m_o = _max3(rp_o, _shl(rp_o, 32), _shl(rp_o, 64))

    def conv2(slices):
        acc = None
        for dy, lhs in enumerate(slices):
            t = jnp.dot(lhs.reshape(4 * B, 704), w2_ref[dy],
                        preferred_element_type=jnp.float32)
            acc = t if acc is None else acc + t
        return _prelu(acc + b2_ref[...], a2_ref[0]).reshape(4, B, 512)

    h2e = conv2([m_e[0:4], m_o[0:4], m_e[1:5]])
    h2o = conv2([m_o[0:4], m_e[1:5], m_o[1:5]])

    r2 = _max3(h2e[0:3], h2o[0:3], h2e[1:4])
    m2 = _max3(r2, _shl(r2, 64), _shl(r2, 128))

    acc3 = None
    for hrow in range(3):
        t = jnp.dot(m2[hrow], w3_ref[hrow], preferred_element_type=jnp.float32)
        acc3 = t if acc3 is None else acc3 + t
    h3 = _prelu(acc3 + b3_ref[...], a3_ref[0])
    h4_ = _prelu(jnp.dot(h3, w4_ref[...], preferred_element_type=jnp.float32)
                 + b4_ref[...], a4_ref[0])
    o_ref[...] = (jnp.dot(h4_, w5_ref[...], preferred_element_type=jnp.float32)
                  + b5_ref[...])


def _banded(sel, wv):
    t = jnp.einsum('wjd,ydco->ywcjo', sel, wv)
    d, wi, ci, jo, co = t.shape
    return t.reshape(d, wi * ci, jo * co)


def _sel(wi_n, jo_n, stride):
    s = np.zeros((wi_n, jo_n, 3), np.float32)
    for j in range(jo_n):
        for dx in range(3):
            s[stride * (j + dx), j, dx] = 1.0
    return s


_SEL1 = _sel(24, 22, 1)
_SEL2 = _sel(22, 8, 2)
_P3 = np.zeros((8, 3), np.float32)
for _w in range(3):
    _P3[2 * _w, _w] = 1.0


@jax.jit
def kernel(x_nchw, w1, b1, w2, b2, w3f, b3f, w4, b4, w5, b5, a1, a2, a3, a4):
    N = x_nchw.shape[0]
    B = 256
    Np = ((N + B - 1) // B) * B
    x = jnp.transpose(x_nchw, (0, 2, 3, 1)).astype(jnp.float32)
    x = jnp.pad(x, ((0, 0), (0, 0), (0, 0), (0, 5))).reshape(N, 24, 192)
    if Np != N:
        x = jnp.pad(x, ((0, Np - N), (0, 0), (0, 0)))
    x = jnp.transpose(x.reshape(Np // B, B, 6, 4, 192), (0, 3, 2, 1, 4))

    w1b = _banded(jnp.asarray(_SEL1), w1.reshape(3, 3, 8, 32))
    w2b = _banded(jnp.asarray(_SEL2), w2.reshape(3, 3, 32, 64))
    w3r = jnp.einsum('vw,hwco->hvco', jnp.asarray(_P3),
                     w3f.reshape(3, 3, 64, 256)).reshape(3, 512, 256)
    b1t = jnp.tile(b1, 22).reshape(1, 704)
    b2t = jnp.tile(b2, 8).reshape(1, 512)

    smem = pl.BlockSpec(memory_space=pltpu.MemorySpace.SMEM)
    full = lambda shape: pl.BlockSpec(shape, lambda i: (0,) * len(shape))
    out = pl.pallas_call(
        _rnet_kernel,
        out_shape=jax.ShapeDtypeStruct((Np, 15), jnp.float32),
        grid=(Np // B,),
        in_specs=[
            pl.BlockSpec((1, 4, 6, B, 192), lambda i: (i, 0, 0, 0, 0)),
            full((3, 192, 704)), full((1, 704)), smem,
            full((3, 704, 512)), full((1, 512)), smem,
            full((3, 512, 256)), full((1, 256)), smem,
            full((256, 128)), full((1, 128)), smem,
            full((128, 15)), full((1, 15)),
        ],
        out_specs=pl.BlockSpec((B, 15), lambda i: (i, 0)),
        compiler_params=pltpu.CompilerParams(
            dimension_semantics=("parallel",)),
    )(x, w1b, b1t, a1, w2b, b2t, a2, w3r, b3f.reshape(1, 256), a3,
      w4, b4.reshape(1, 128), a4, w5, b5.reshape(1, 15))
    return out[:N] if Np != N else out


# speedup vs baseline: 348.8786x; 1.2682x over previous
import jax
import jax.numpy as jnp
import numpy as np
from jax import lax
from jax.experimental import pallas as pl
from jax.experimental.pallas import tpu as pltpu


def _prelu(y, a):
    return jnp.where(y > 0, y, y * a)


def _max3(a, b, c):
    return jnp.maximum(jnp.maximum(a, b), c)


def _shl(v, lanes):
    n = v.shape[-1]
    return jnp.pad(lax.slice_in_dim(v, lanes, n, axis=2),
                   ((0, 0), (0, 0), (0, lanes)))


def _rnet_kernel(x_ref, w1_ref, b1_ref, a1_ref, w2_ref, b2_ref, a2_ref,
                 w3_ref, b3_ref, a3_ref, w4_ref, b4_ref, a4_ref,
                 w5_ref, b5_ref, o_ref):
    B = x_ref.shape[3]
    xq = [x_ref[0, q] for q in range(4)]

    h4 = []
    for q in range(4):
        nrows = 6 if q < 2 else 5
        acc = None
        for dy in range(3):
            r = q + dy
            lhs = xq[r % 4][r // 4:r // 4 + nrows].reshape(nrows * B, 72)
            t = jnp.dot(lhs, w1_ref[dy], preferred_element_type=jnp.float32)
            acc = t if acc is None else acc + t
        h4.append(_prelu(acc + b1_ref[...], a1_ref[0]).reshape(nrows, B, 704))

    rp_e = _max3(h4[0][:5], h4[1][:5], h4[2][:5])
    rp_o = _max3(h4[2][:5], h4[3][:5], h4[0][1:6])
    m_e = _max3(rp_e, _shl(rp_e, 32), _shl(rp_e, 64))
    m_o = _max3(rp_o, _shl(rp_o, 32), _shl(rp_o, 64))

    def conv2(slices):
        acc = None
        for dy, lhs in enumerate(slices):
            t = jnp.dot(lhs.reshape(4 * B, 704), w2_ref[dy],
                        preferred_element_type=jnp.float32)
            acc = t if acc is None else acc + t
        return _prelu(acc + b2_ref[...], a2_ref[0]).reshape(4, B, 512)

    h2e = conv2([m_e[0:4], m_o[0:4], m_e[1:5]])
    h2o = conv2([m_o[0:4], m_e[1:5], m_o[1:5]])

    r2 = _max3(h2e[0:3], h2o[0:3], h2e[1:4])
    m2 = _max3(r2, _shl(r2, 64), _shl(r2, 128))

    acc3 = None
    for hrow in range(3):
        t = jnp.dot(m2[hrow], w3_ref[hrow], preferred_element_type=jnp.float32)
        acc3 = t if acc3 is None else acc3 + t
    h3 = _prelu(acc3 + b3_ref[...], a3_ref[0])
    h4_ = _prelu(jnp.dot(h3, w4_ref[...], preferred_element_type=jnp.float32)
                 + b4_ref[...], a4_ref[0])
    o_ref[...] = (jnp.dot(h4_, w5_ref[...], preferred_element_type=jnp.float32)
                  + b5_ref[...])


def _banded(sel, wv):
    t = jnp.einsum('wjd,ydco->ywcjo', sel, wv)
    d, wi, ci, jo, co = t.shape
    return t.reshape(d, wi * ci, jo * co)


def _sel(wi_n, jo_n, stride):
    s = np.zeros((wi_n, jo_n, 3), np.float32)
    for j in range(jo_n):
        for dx in range(3):
            s[stride * (j + dx), j, dx] = 1.0
    return s


_SEL1 = _sel(24, 22, 1)
_SEL2 = _sel(22, 8, 2)
_P3 = np.zeros((8, 3), np.float32)
for _w in range(3):
    _P3[2 * _w, _w] = 1.0


@jax.jit
def kernel(x_nchw, w1, b1, w2, b2, w3f, b3f, w4, b4, w5, b5, a1, a2, a3, a4):
    N = x_nchw.shape[0]
    B = 256
    Np = ((N + B - 1) // B) * B
    x = x_nchw.astype(jnp.float32)
    if Np != N:
        x = jnp.pad(x, ((0, Np - N), (0, 0), (0, 0), (0, 0)))
    x = jnp.transpose(x.reshape(Np // B, B, 3, 6, 4, 24),
                      (0, 4, 3, 1, 2, 5)).reshape(Np // B, 4, 6, B, 72)

    w1b = jnp.einsum('wjd,ydco->ycwjo', jnp.asarray(_SEL1),
                     w1.reshape(3, 3, 8, 32)[:, :, :3, :]).reshape(3, 72, 704)
    w2b = _banded(jnp.asarray(_SEL2), w2.reshape(3, 3, 32, 64))
    w3r = jnp.einsum('vw,hwco->hvco', jnp.asarray(_P3),
                     w3f.reshape(3, 3, 64, 256)).reshape(3, 512, 256)
    b1t = jnp.tile(b1, 22).reshape(1, 704)
    b2t = jnp.tile(b2, 8).reshape(1, 512)

    smem = pl.BlockSpec(memory_space=pltpu.MemorySpace.SMEM)
    full = lambda shape: pl.BlockSpec(shape, lambda i: (0,) * len(shape))
    out = pl.pallas_call(
        _rnet_kernel,
        out_shape=jax.ShapeDtypeStruct((Np, 15), jnp.float32),
        grid=(Np // B,),
        in_specs=[
            pl.BlockSpec((1, 4, 6, B, 72), lambda i: (i, 0, 0, 0, 0)),
            full((3, 72, 704)), full((1, 704)), smem,
            full((3, 704, 512)), full((1, 512)), smem,
            full((3, 512, 256)), full((1, 256)), smem,
            full((256, 128)), full((1, 128)), smem,
            full((128, 15)), full((1, 15)),
        ],
        out_specs=pl.BlockSpec((B, 15), lambda i: (i, 0)),
        compiler_params=pltpu.CompilerParams(
            dimension_semantics=("parallel",)),
    )(x, w1b, b1t, a1, w2b, b2t, a2, w3r, b3f.reshape(1, 256), a3,
      w4, b4.reshape(1, 128), a4, w5, b5.reshape(1, 15))
    return out[:N] if Np != N else out
